# baseline (speedup 1.0000x reference)
import functools

import jax
import jax.numpy as jnp
import numpy as np
from jax import lax
from jax.experimental import pallas as pl
from jax.experimental.pallas import tpu as pltpu
from jax.experimental.pallas import tpu_sc as plsc

NUM_FEATURES = 21
VOCAB = 2000
EMB_DIM = 128
LANES = 16
NUM_CORES = 2
NUM_SUBCORES = 16
NUM_WORKERS = NUM_CORES * NUM_SUBCORES
B_STEP = 16
WORDS = EMB_DIM // 2
WREGS = WORDS // LANES
NBUF = 4
STEP_IDX = NUM_FEATURES * B_STEP
QUAD_INTS = 4 * STEP_IDX
N_STREAMS = 3
IDX_PER_STREAM = STEP_IDX // N_STREAMS


def _make_sc_call(n_pad):
    rows_per_tile = n_pad // NUM_WORKERS
    steps = rows_per_tile // B_STEP
    assert steps % 4 == 0 and steps >= 8
    nquads = steps // 4
    mesh = plsc.VectorSubcoreMesh(core_axis_name="c", subcore_axis_name="s")

    @functools.partial(
        pl.kernel,
        out_type=jax.ShapeDtypeStruct((n_pad, EMB_DIM), jnp.float32),
        mesh=mesh,
        scratch_types=[
            pltpu.VMEM((QUAD_INTS,), jnp.int32),
            pltpu.VMEM((QUAD_INTS,), jnp.int32),
            pltpu.VMEM((NBUF, STEP_IDX), jnp.int32),
            pltpu.VMEM((NBUF, STEP_IDX, WORDS), jnp.int32),
            pltpu.VMEM((NBUF, B_STEP, EMB_DIM), jnp.float32),
            pltpu.SemaphoreType.DMA,
            pltpu.SemaphoreType.DMA,
            pltpu.SemaphoreType.DMA,
            pltpu.SemaphoreType.DMA,
            pltpu.SemaphoreType.DMA,
            pltpu.SemaphoreType.DMA,
            pltpu.SemaphoreType.DMA,
            pltpu.SemaphoreType.DMA,
            pltpu.SemaphoreType.DMA,
            pltpu.SemaphoreType.DMA,
        ],
        compiler_params=pltpu.CompilerParams(
            needs_layout_passes=False, use_tc_tiling_on_sc=False
        ),
    )
    def sc_kernel(
        x_hbm, tab_hbm, out_hbm,
        xbuf0, xbuf1, ibuf, gbuf, obuf,
        xsem0, xsem1,
        gsem0, gsem1, gsem2, gsem3,
        osem0, osem1, osem2, osem3,
    ):
        wid = lax.axis_index("s") * NUM_CORES + lax.axis_index("c")
        tile_base = wid * rows_per_tile
        base_ids = lax.iota(jnp.int32, LANES) * NUM_FEATURES
        xbufs = (xbuf0, xbuf1)
        xsems = (xsem0, xsem1)
        gsems = (gsem0, gsem1, gsem2, gsem3)
        osems = (osem0, osem1, osem2, osem3)

        def xload(q, xb):
            @pl.when(q < nquads)
            def _():
                base = tile_base + q * 4 * B_STEP
                pltpu.async_copy(
                    x_hbm.at[pl.ds(base * NUM_FEATURES, QUAD_INTS)],
                    xbufs[xb],
                    xsems[xb],
                )

        def xwait(xb):
            pltpu.make_async_copy(
                x_hbm.at[pl.ds(0, QUAD_INTS)], xbufs[xb], xsems[xb]
            ).wait()

        def fire(s, b, xb, qoff):
            @pl.when(s < steps)
            def _():
                for i in range(NUM_FEATURES):
                    src = base_ids + (qoff * STEP_IDX + i)
                    idx = plsc.load_gather(xbufs[xb], [src])
                    ibuf[b, pl.ds(i * B_STEP, LANES)] = idx + (i * VOCAB)
                for k in range(N_STREAMS):
                    sl = pl.ds(k * IDX_PER_STREAM, IDX_PER_STREAM)
                    pltpu.async_copy(
                        tab_hbm.at[ibuf.at[b, sl]], gbuf.at[b, sl, :], gsems[b]
                    )

        def consume(s, b):
            @pl.when(s < steps)
            def _():
                base = tile_base + s * B_STEP
                pltpu.make_async_copy(
                    tab_hbm.at[pl.ds(0, STEP_IDX), :], gbuf.at[b], gsems[b]
                ).wait()
                @pl.when(s >= NBUF)
                def _():
                    pltpu.make_async_copy(
                        obuf.at[b], out_hbm.at[pl.ds(base, B_STEP), :], osems[b]
                    ).wait()

                @plsc.parallel_loop(0, B_STEP)
                def _(j):
                    for k in range(WREGS):
                        sl = pl.ds(k * LANES, LANES)
                        w = gbuf[b, j, sl]
                        acc_e = plsc.bitcast(lax.shift_left(w, 16), jnp.float32)
                        acc_o = plsc.bitcast(w, jnp.float32)
                        for i in range(1, NUM_FEATURES):
                            w = gbuf[b, i * B_STEP + j, sl]
                            acc_e = acc_e + plsc.bitcast(
                                lax.shift_left(w, 16), jnp.float32
                            )
                            acc_o = acc_o + plsc.bitcast(w, jnp.float32)
                        obuf[b, j, pl.ds(2 * k * LANES, LANES)] = acc_e
                        obuf[b, j, pl.ds((2 * k + 1) * LANES, LANES)] = acc_o

                pltpu.async_copy(
                    obuf.at[b], out_hbm.at[pl.ds(base, B_STEP), :], osems[b]
                )

        pltpu.sync_copy(x_hbm.at[pl.ds(tile_base * NUM_FEATURES, QUAD_INTS)], xbuf0)
        xload(1, 1)
        fire(0, 0, 0, 0)
        fire(1, 1, 0, 1)
        fire(2, 2, 0, 2)

        def body2(h, carry):
            for par in range(2):
                g = h * 2 + par
                xc = par
                xo = 1 - par
                s = g * 4
                consume(s, 0)
                fire(s + 3, 3, xc, 3)
                xload(g + 2, xc)
                consume(s + 1, 1)

                @pl.when(g + 1 < nquads)
                def _():
                    xwait(xo)

                fire(s + 4, 0, xo, 0)
                consume(s + 2, 2)
                fire(s + 5, 1, xo, 1)
                consume(s + 3, 3)
                fire(s + 6, 2, xo, 2)
            return carry

        lax.fori_loop(0, (nquads + 1) // 2, body2, 0, unroll=False)
        for b in range(NBUF):
            pltpu.make_async_copy(
                obuf.at[b], out_hbm.at[pl.ds(0, B_STEP), :], osems[b]
            ).wait()

    return sc_kernel


_COLPERM = np.empty(EMB_DIM, np.int32)
for _g in range(WREGS):
    for _t in range(LANES):
        _COLPERM[32 * _g + 2 * _t] = 32 * _g + _t
        _COLPERM[32 * _g + 2 * _t + 1] = 32 * _g + LANES + _t


def kernel(x, tables):
    n = x.shape[0]
    block = NUM_WORKERS * B_STEP * 4
    n_pad = ((n + block - 1) // block) * block
    if n_pad != n:
        x = jnp.pad(x, ((0, n_pad - n), (0, 0)))
    tab = tables.reshape(NUM_FEATURES * VOCAB, EMB_DIM)
    tab = tab[:, _COLPERM].astype(jnp.bfloat16)
    tab_i32 = lax.bitcast_convert_type(
        tab.reshape(NUM_FEATURES * VOCAB, WORDS, 2), jnp.int32
    )
    out = _make_sc_call(n_pad)(x.reshape(-1), tab_i32)
    return out[:n]


# speedup vs baseline: 1.0733x; 1.0733x over previous
import functools

import jax
import jax.numpy as jnp
import numpy as np
from jax import lax
from jax.experimental import pallas as pl
from jax.experimental.pallas import tpu as pltpu
from jax.experimental.pallas import tpu_sc as plsc

NUM_FEATURES = 21
VOCAB = 2000
EMB_DIM = 128
LANES = 16
NUM_CORES = 2
NUM_SUBCORES = 16
NUM_WORKERS = NUM_CORES * NUM_SUBCORES
B_STEP = 32
WORDS = EMB_DIM // 2
WREGS = WORDS // LANES
PAIR_INTS = 2 * B_STEP * NUM_FEATURES
STEP_IDX = NUM_FEATURES * B_STEP
N_STREAMS = 6
IDX_PER_STREAM = STEP_IDX // N_STREAMS


def _make_sc_call(n_pad):
    rows_per_tile = n_pad // NUM_WORKERS
    steps = rows_per_tile // B_STEP
    assert steps % 2 == 0 and steps >= 4
    npairs = steps // 2
    mesh = plsc.VectorSubcoreMesh(core_axis_name="c", subcore_axis_name="s")

    @functools.partial(
        pl.kernel,
        out_type=jax.ShapeDtypeStruct((n_pad, EMB_DIM), jnp.float32),
        mesh=mesh,
        scratch_types=[
            pltpu.VMEM((PAIR_INTS,), jnp.int32),
            pltpu.VMEM((PAIR_INTS,), jnp.int32),
            pltpu.VMEM((STEP_IDX,), jnp.int32),
            pltpu.VMEM((STEP_IDX,), jnp.int32),
            pltpu.VMEM((2, STEP_IDX, WORDS), jnp.int32),
            pltpu.VMEM((2, B_STEP, EMB_DIM), jnp.float32),
            pltpu.SemaphoreType.DMA,
            pltpu.SemaphoreType.DMA,
            pltpu.SemaphoreType.DMA,
            pltpu.SemaphoreType.DMA,
            pltpu.SemaphoreType.DMA,
            pltpu.SemaphoreType.DMA,
        ],
        compiler_params=pltpu.CompilerParams(
            needs_layout_passes=False, use_tc_tiling_on_sc=False
        ),
    )
    def sc_kernel(
        x_hbm, tab_hbm, out_hbm,
        xbuf0, xbuf1, ibuf0, ibuf1, gbuf, obuf,
        xsem0, xsem1, gsem0, gsem1, osem0, osem1,
    ):
        wid = lax.axis_index("s") * NUM_CORES + lax.axis_index("c")
        tile_base = wid * rows_per_tile
        base_ids = lax.iota(jnp.int32, LANES) * NUM_FEATURES
        xbufs = (xbuf0, xbuf1)
        ibufs = (ibuf0, ibuf1)
        xsems = (xsem0, xsem1)
        gsems = (gsem0, gsem1)
        osems = (osem0, osem1)

        def xload(p, xb):
            base = tile_base + p * 2 * B_STEP
            pltpu.async_copy(
                x_hbm.at[pl.ds(base * NUM_FEATURES, PAIR_INTS)], xbufs[xb], xsems[xb]
            )

        def xwait(xb):
            pltpu.make_async_copy(
                x_hbm.at[pl.ds(0, PAIR_INTS)], xbufs[xb], xsems[xb]
            ).wait()

        def fire(s, b, xb, xoff):
            ibuf = ibufs[b]
            for i in range(NUM_FEATURES):
                for h in range(B_STEP // LANES):
                    src = base_ids + (xoff + h * LANES * NUM_FEATURES + i)
                    idx = plsc.load_gather(xbufs[xb], [src])
                    ibuf[pl.ds(i * B_STEP + h * LANES, LANES)] = idx + (i * VOCAB)
            for k in range(N_STREAMS):
                sl = pl.ds(k * IDX_PER_STREAM, IDX_PER_STREAM)
                pltpu.async_copy(tab_hbm.at[ibuf.at[sl]], gbuf.at[b, sl, :], gsems[b])

        def consume(s, b):
            base = tile_base + s * B_STEP
            pltpu.make_async_copy(
                tab_hbm.at[pl.ds(0, STEP_IDX), :], gbuf.at[b], gsems[b]
            ).wait()
            @pl.when(s >= 2)
            def _():
                pltpu.make_async_copy(
                    obuf.at[b], out_hbm.at[pl.ds(base, B_STEP), :], osems[b]
                ).wait()

            @plsc.parallel_loop(0, B_STEP)
            def _(j):
                for k in range(WREGS):
                    sl = pl.ds(k * LANES, LANES)
                    w = gbuf[b, j, sl]
                    acc_e = plsc.bitcast(lax.shift_left(w, 16), jnp.float32)
                    acc_o = plsc.bitcast(w, jnp.float32)
                    for i in range(1, NUM_FEATURES):
                        w = gbuf[b, i * B_STEP + j, sl]
                        acc_e = acc_e + plsc.bitcast(
                            lax.shift_left(w, 16), jnp.float32
                        )
                        acc_o = acc_o + plsc.bitcast(w, jnp.float32)
                    obuf[b, j, pl.ds(2 * k * LANES, LANES)] = acc_e
                    obuf[b, j, pl.ds((2 * k + 1) * LANES, LANES)] = acc_o

            pltpu.async_copy(obuf.at[b], out_hbm.at[pl.ds(base, B_STEP), :], osems[b])

        pltpu.sync_copy(x_hbm.at[pl.ds(tile_base * NUM_FEATURES, PAIR_INTS)], xbuf0)
        fire(0, 0, 0, 0)

        def quad_body(q, carry):
            for u in range(2):
                p = q * 2 + u
                xb = u
                nxb = 1 - u
                s = p * 2

                @pl.when(p < npairs)
                def _():
                    @pl.when(p + 1 < npairs)
                    def _():
                        xload(p + 1, nxb)

                    fire(s + 1, 1, xb, B_STEP * NUM_FEATURES)
                    consume(s, 0)

                    @pl.when(p + 1 < npairs)
                    def _():
                        xwait(nxb)
                        fire(s + 2, 0, nxb, 0)

                    consume(s + 1, 1)

            return carry

        lax.fori_loop(0, (npairs + 1) // 2, quad_body, 0, unroll=False)
        for b in range(2):
            pltpu.make_async_copy(
                obuf.at[b], out_hbm.at[pl.ds(0, B_STEP), :], osems[b]
            ).wait()

    return sc_kernel


_COLPERM = np.empty(EMB_DIM, np.int32)
for _g in range(WREGS):
    for _t in range(LANES):
        _COLPERM[32 * _g + 2 * _t] = 32 * _g + _t
        _COLPERM[32 * _g + 2 * _t + 1] = 32 * _g + LANES + _t


def kernel(x, tables):
    n = x.shape[0]
    block = NUM_WORKERS * B_STEP * 2
    n_pad = ((n + block - 1) // block) * block
    if n_pad != n:
        x = jnp.pad(x, ((0, n_pad - n), (0, 0)))
    tab = tables.reshape(NUM_FEATURES * VOCAB, EMB_DIM)
    tab = tab[:, _COLPERM].astype(jnp.bfloat16)
    tab_i32 = lax.bitcast_convert_type(
        tab.reshape(NUM_FEATURES * VOCAB, WORDS, 2), jnp.int32
    )
    out = _make_sc_call(n_pad)(x.reshape(-1), tab_i32)
    return out[:n]
